# Initial kernel scaffold; baseline (speedup 1.0000x reference)
import jax
import jax.numpy as jnp
from jax.experimental import pallas as pl


def kernel(preds, targets, paths):
    raise NotImplementedError("write your pallas kernel here")


# speedup vs baseline: 111.0133x; 111.0133x over previous
import jax
import jax.numpy as jnp
from jax import lax
from jax.experimental import pallas as pl
from jax.experimental.pallas import tpu as pltpu
from jax.experimental.pallas import tpu_sc as plsc

_B, _S, _P = 16, 4096, 8192
_NC, _NS, _L = 2, 16, 16
_NW = _NC * _NS
_PPW = _B * _P // _NW
_ITERS = _PPW // _L


def _dtw_body(preds_hbm, targets_hbm, iidx_hbm, jidx_hbm, out_hbm,
              preds_v, targs_v, iidx_v, jidx_v, acc_v):
    wid = lax.axis_index("s") * _NC + lax.axis_index("c")
    b = wid // 2
    base = wid * _PPW

    pltpu.sync_copy(preds_hbm.at[b], preds_v)
    pltpu.sync_copy(targets_hbm.at[b], targs_v)
    pltpu.sync_copy(iidx_hbm.at[pl.ds(base, _PPW)], iidx_v)
    pltpu.sync_copy(jidx_hbm.at[pl.ds(base, _PPW)], jidx_v)

    def step(k, acc):
        iv = iidx_v[pl.ds(k * _L, _L)]
        jv = jidx_v[pl.ds(k * _L, _L)]
        i2 = iv * 2
        j2 = jv * 2
        px = plsc.load_gather(preds_v, [i2])
        py = plsc.load_gather(preds_v, [i2 + 1])
        tx = plsc.load_gather(targs_v, [j2])
        ty = plsc.load_gather(targs_v, [j2 + 1])
        return acc + (jnp.abs(px - tx) + jnp.abs(py - ty))

    acc = lax.fori_loop(0, _ITERS, step, jnp.zeros((_L,), jnp.float32))
    acc_v[...] = acc
    pltpu.sync_copy(acc_v, out_hbm.at[wid])


def kernel(preds, targets, paths):
    preds2 = preds.reshape(_B, _S * 2)
    targets2 = targets.reshape(_B, _S * 2)
    iidx = paths[..., 0].reshape(_B * _P)
    jidx = paths[..., 1].reshape(_B * _P)
    partials = pl.kernel(
        _dtw_body,
        out_type=jax.ShapeDtypeStruct((_NW, _L), jnp.float32),
        mesh=plsc.VectorSubcoreMesh(core_axis_name="c", subcore_axis_name="s"),
        compiler_params=pltpu.CompilerParams(needs_layout_passes=False),
        scratch_types=[
            pltpu.VMEM((_S * 2,), jnp.float32),
            pltpu.VMEM((_S * 2,), jnp.float32),
            pltpu.VMEM((_PPW,), jnp.int32),
            pltpu.VMEM((_PPW,), jnp.int32),
            pltpu.VMEM((_L,), jnp.float32),
        ],
    )(preds2, targets2, iidx, jidx)
    return jnp.sum(partials) / (_B * _S)
